# baseline (speedup 1.0000x reference)
import jax
import jax.numpy as jnp
from jax.experimental import pallas as pl
from jax.experimental.pallas import tpu as pltpu

TILE_A = 256
TILE_B = 512


def _fused_kernel(a_ref, xb_ref, out_ref, asc_s, y1b_s):
    n = asc_s.shape[0]
    nt_a = n // TILE_A
    d = xb_ref.shape[1]
    i = pl.program_id(0)

    @pl.when(i < nt_a)
    def _():
        a = a_ref[...]
        deg = jnp.maximum(jnp.sum(a, axis=1, keepdims=True), 1)
        r = 1.0 / deg.astype(jnp.float32)
        asc = (a.astype(jnp.float32) * r).astype(jnp.bfloat16)
        asc_s[pl.ds(i * TILE_A, TILE_A), :] = asc
        y1 = jnp.dot(asc, xb_ref[...], preferred_element_type=jnp.float32)
        y1b_s[pl.ds(i * TILE_A, TILE_A), :] = y1.astype(jnp.bfloat16)

    @pl.when(i >= nt_a)
    def _():
        j = i - nt_a
        y1f = y1b_s[pl.ds(j * TILE_B, TILE_B), :].astype(jnp.float32)
        out_ref[:, 0:d] = xb_ref[pl.ds(j * TILE_B, TILE_B), :].astype(jnp.float32)
        out_ref[:, d:2 * d] = y1f
        out_ref[:, 2 * d:3 * d] = y1f
        y2 = jnp.dot(
            asc_s[pl.ds(j * TILE_B, TILE_B), :],
            y1b_s[...],
            preferred_element_type=jnp.float32,
        )
        out_ref[:, 3 * d:4 * d] = y2


def kernel(adjacency_matrix, node_features):
    n, d = node_features.shape
    nt_a = n // TILE_A
    nt_b = n // TILE_B
    xb = node_features.astype(jnp.bfloat16)

    out = pl.pallas_call(
        _fused_kernel,
        grid=(nt_a + nt_b,),
        in_specs=[
            pl.BlockSpec((TILE_A, n), lambda i: (jnp.minimum(i, nt_a - 1), 0)),
            pl.BlockSpec((n, d), lambda i: (0, 0)),
        ],
        out_specs=pl.BlockSpec(
            (TILE_B, 4 * d), lambda i: (jnp.maximum(i - nt_a, 0), 0)
        ),
        out_shape=jax.ShapeDtypeStruct((n, 4 * d), jnp.float32),
        scratch_shapes=[
            pltpu.VMEM((n, n), jnp.bfloat16),
            pltpu.VMEM((n, d), jnp.bfloat16),
        ],
        compiler_params=pltpu.CompilerParams(
            dimension_semantics=("arbitrary",),
        ),
    )(adjacency_matrix, xb)

    return out


# speedup vs baseline: 1.1066x; 1.1066x over previous
import jax
import jax.numpy as jnp
from jax.experimental import pallas as pl
from jax.experimental.pallas import tpu as pltpu

TILE_A = 512
TILE_B = 1024


def _fused_kernel(a_ref, xb_ref, out_ref, a8_s, y1b_s, recip_s):
    n = a8_s.shape[0]
    nt_a = n // TILE_A
    d = xb_ref.shape[1]
    i = pl.program_id(0)

    @pl.when(i < nt_a)
    def _():
        a = a_ref[...]
        a8 = a.astype(jnp.int8)
        a8_s[pl.ds(i * TILE_A, TILE_A), :] = a8
        deg = jnp.maximum(jnp.sum(a, axis=1, keepdims=True), 1)
        r = 1.0 / deg.astype(jnp.float32)
        recip_s[pl.ds(i * TILE_A, TILE_A), :] = r
        ab = a8.astype(jnp.bfloat16)
        y1 = jnp.dot(ab, xb_ref[...], preferred_element_type=jnp.float32) * r
        y1b_s[pl.ds(i * TILE_A, TILE_A), :] = y1.astype(jnp.bfloat16)

    @pl.when(i >= nt_a)
    def _():
        j = i - nt_a
        y1f = y1b_s[pl.ds(j * TILE_B, TILE_B), :].astype(jnp.float32)
        out_ref[:, 0:d] = xb_ref[pl.ds(j * TILE_B, TILE_B), :].astype(jnp.float32)
        out_ref[:, d:2 * d] = y1f
        out_ref[:, 2 * d:3 * d] = y1f
        ab = a8_s[pl.ds(j * TILE_B, TILE_B), :].astype(jnp.bfloat16)
        r = recip_s[pl.ds(j * TILE_B, TILE_B), :]
        y2 = jnp.dot(ab, y1b_s[...], preferred_element_type=jnp.float32) * r
        out_ref[:, 3 * d:4 * d] = y2


def kernel(adjacency_matrix, node_features):
    n, d = node_features.shape
    nt_a = n // TILE_A
    nt_b = n // TILE_B
    xb = node_features.astype(jnp.bfloat16)

    out = pl.pallas_call(
        _fused_kernel,
        grid=(nt_a + nt_b,),
        in_specs=[
            pl.BlockSpec((TILE_A, n), lambda i: (jnp.minimum(i, nt_a - 1), 0)),
            pl.BlockSpec((n, d), lambda i: (0, 0)),
        ],
        out_specs=pl.BlockSpec(
            (TILE_B, 4 * d), lambda i: (jnp.maximum(i - nt_a, 0), 0)
        ),
        out_shape=jax.ShapeDtypeStruct((n, 4 * d), jnp.float32),
        scratch_shapes=[
            pltpu.VMEM((n, n), jnp.int8),
            pltpu.VMEM((n, d), jnp.bfloat16),
            pltpu.VMEM((n, 1), jnp.float32),
        ],
        compiler_params=pltpu.CompilerParams(
            dimension_semantics=("arbitrary",),
        ),
    )(adjacency_matrix, xb)

    return out
